# baseline (speedup 1.0000x reference)
import functools
import math

import jax
import jax.numpy as jnp
from jax import lax
from jax.experimental import pallas as pl
from jax.experimental.pallas import tpu as pltpu
from jax.experimental.pallas import tpu_sc as plsc

D_MODEL = 64
SCALE = math.sqrt(D_MODEL)
VOCAB = 1_000_000

NC = 2
NS = 16
NW = NC * NS
LANES = 16

NR_FULL = VOCAB // 128
RPW = NR_FULL // NW
EXTRA = NR_FULL - NW * RPW
TAIL_ROWS = VOCAB - NR_FULL * 128
ABUF = 4

N_I1 = 200
BATCH = 128
BBUF = 4


def _detile_body(lutT, tail, lin, tbuf, obuf, tailv, *sems):
    gsems = sems[:ABUF]
    ssems = sems[ABUF:]
    wid = lax.axis_index("s") * NC + lax.axis_index("c")
    base = wid * RPW
    idx16 = lax.iota(jnp.int32, 16)

    def start_fetch(tr, b):
        pltpu.async_copy(lutT.at[:, pl.ds(tr * 128, 128)], tbuf.at[b],
                         gsems[b])

    def wait_fetch(tr, b):
        pltpu.make_async_copy(lutT.at[:, pl.ds(tr * 128, 128)], tbuf.at[b],
                              gsems[b]).wait()

    def start_write(tr, b):
        pltpu.async_copy(obuf.at[b], lin.at[pl.ds(tr * 64, 64), :],
                         ssems[b])

    def wait_write(tr, b):
        pltpu.make_async_copy(obuf.at[b], lin.at[pl.ds(tr * 64, 64), :],
                              ssems[b]).wait()

    def transpose_scale(b):
        @pl.loop(0, 128, unroll=8)
        def _rows(r):
            for m in range(D_MODEL // LANES):
                v = plsc.load_gather(
                    tbuf.at[b],
                    [idx16 + (16 * m), jnp.full((16,), r, jnp.int32)])
                obuf[b, r // 2, pl.ds((r % 2) * 64 + 16 * m, 16)] = v * SCALE

    for b in range(ABUF):
        start_fetch(base + b, b)

    for b in range(ABUF):
        tr = base + b
        wait_fetch(tr, b)
        transpose_scale(b)
        start_write(tr, b)
        start_fetch(tr + ABUF, b)

    @pl.loop(ABUF, RPW - ABUF, step=ABUF)
    def _main(g):
        for b in range(ABUF):
            tr = base + g + b
            wait_fetch(tr, b)
            wait_write(tr - ABUF, b)
            transpose_scale(b)
            start_write(tr, b)
            start_fetch(tr + ABUF, b)

    for b in range(ABUF):
        tr = base + RPW - ABUF + b
        wait_fetch(tr, b)
        wait_write(tr - ABUF, b)
        transpose_scale(b)
        start_write(tr, b)
    for b in range(ABUF):
        wait_write(base + RPW - ABUF + b, b)

    @pl.when(wid < EXTRA)
    def _extra():
        trx = NW * RPW + wid
        pltpu.sync_copy(lutT.at[:, pl.ds(trx * 128, 128)], tbuf.at[0])
        transpose_scale(0)
        pltpu.sync_copy(obuf.at[0], lin.at[pl.ds(trx * 64, 64), :])

    @pl.when(wid == EXTRA)
    def _tail():
        pltpu.sync_copy(tail, tailv)
        pltpu.sync_copy(tailv, lin.at[pl.ds(NR_FULL * 64, TAIL_ROWS // 2), :])


def _gather_body(x6, lin, q, xv, idx2, gbuf, qbuf, *sems):
    gsems = sems[:BBUF]
    ssems = sems[BBUF:]
    wid = lax.axis_index("s") * NC + lax.axis_index("c")
    idx16 = lax.iota(jnp.int32, 16)

    pltpu.sync_copy(x6.at[wid], xv)

    def prep_lines(j, b):
        for m in range(BATCH // LANES):
            iv = xv[j // 8, j % 8, pl.ds(16 * m, 16)]
            idx2[b, pl.ds(16 * m, 16)] = lax.shift_right_logical(iv, 1)

    def start_gather(j, b):
        prep_lines(j, b)
        pltpu.async_copy(lin.at[idx2.at[b]], gbuf.at[b], gsems[b])

    def wait_gather(j, b):
        pltpu.make_async_copy(lin.at[idx2.at[b]], gbuf.at[b],
                              gsems[b]).wait()

    def q_slice(j):
        return q.at[j, :, pl.ds(wid * 128, 128)]

    def start_scatter(j, b):
        pltpu.async_copy(qbuf.at[b], q_slice(j), ssems[b])

    def wait_scatter(j, b):
        pltpu.make_async_copy(qbuf.at[b], q_slice(j), ssems[b]).wait()

    def transpose_block(j, b):
        paroff = []
        for m in range(BATCH // LANES):
            iv = xv[j // 8, j % 8, pl.ds(16 * m, 16)]
            paroff.append(lax.shift_left(iv & 1, 6))

        @pl.loop(0, D_MODEL, unroll=4)
        def _cols(d):
            for m in range(BATCH // LANES):
                v = plsc.load_gather(gbuf.at[b],
                                     [idx16 + (16 * m), paroff[m] + d])
                qbuf[b, d, pl.ds(16 * m, 16)] = v

    for b in range(BBUF):
        start_gather(b, b)

    for b in range(BBUF):
        wait_gather(b, b)
        transpose_block(b, b)
        start_scatter(b, b)
        start_gather(b + BBUF, b)

    @pl.loop(BBUF, N_I1 - BBUF, step=BBUF)
    def _main(g):
        for b in range(BBUF):
            j = g + b
            wait_gather(j, b)
            wait_scatter(j - BBUF, b)
            transpose_block(j, b)
            start_scatter(j, b)
            start_gather(j + BBUF, b)

    for b in range(BBUF):
        j = N_I1 - BBUF + b
        wait_gather(j, b)
        wait_scatter(j - BBUF, b)
        transpose_block(j, b)
        start_scatter(j, b)
    for b in range(BBUF):
        wait_scatter(N_I1 - BBUF + b, b)


def kernel(x, lut):
    rows, cols = x.shape
    assert (rows, cols) == (4096, N_I1)
    assert lut.shape == (VOCAB, D_MODEL)

    mesh = plsc.VectorSubcoreMesh(
        core_axis_name="c", subcore_axis_name="s",
        num_cores=NC, num_subcores=NS)

    lutT = lut.T
    tail = (lut[NR_FULL * 128:] * SCALE).reshape(TAIL_ROWS // 2, 128)
    detile = pl.kernel(
        _detile_body,
        out_type=jax.ShapeDtypeStruct((VOCAB // 2, 128), jnp.float32),
        mesh=mesh,
        scratch_types=(
            [pltpu.VMEM((ABUF, 64, 128), jnp.float32),
             pltpu.VMEM((ABUF, 64, 128), jnp.float32),
             pltpu.VMEM((TAIL_ROWS // 2, 128), jnp.float32)]
            + [pltpu.SemaphoreType.DMA] * (2 * ABUF)
        ),
        compiler_params=pltpu.CompilerParams(
            use_tc_tiling_on_sc=True, needs_layout_passes=False),
    )
    lin = detile(lutT, tail)

    x6 = x.T.reshape(25, 8, 32, 128).transpose(2, 0, 1, 3)
    gather = pl.kernel(
        _gather_body,
        out_type=jax.ShapeDtypeStruct((N_I1, D_MODEL, 4096), jnp.float32),
        mesh=mesh,
        scratch_types=(
            [pltpu.VMEM((25, 8, 128), jnp.int32),
             pltpu.VMEM((BBUF, 128), jnp.int32),
             pltpu.VMEM((BBUF, BATCH, 128), jnp.float32),
             pltpu.VMEM((BBUF, D_MODEL, 128), jnp.float32)]
            + [pltpu.SemaphoreType.DMA] * (2 * BBUF)
        ),
        compiler_params=pltpu.CompilerParams(
            use_tc_tiling_on_sc=True, needs_layout_passes=False),
    )
    q = gather(x6, lin)

    return q.transpose(2, 0, 1)


# speedup vs baseline: 1.2622x; 1.2622x over previous
import functools
import math

import jax
import jax.numpy as jnp
from jax import lax
from jax.experimental import pallas as pl
from jax.experimental.pallas import tpu as pltpu
from jax.experimental.pallas import tpu_sc as plsc

D_MODEL = 64
SCALE = math.sqrt(D_MODEL)
VOCAB = 1_000_000

NC = 2
NS = 16
NW = NC * NS
LANES = 16

NR_FULL = VOCAB // 128
RPW = NR_FULL // NW
EXTRA = NR_FULL - NW * RPW
TAIL_ROWS = VOCAB - NR_FULL * 128
ABUF = 4

N_I1 = 200
BATCH = 128
BBUF = 4


def _detile_body(lutT, tail, lin, tbuf, obuf, tailv, *sems):
    gsems = sems[:ABUF]
    ssems = sems[ABUF:]
    wid = lax.axis_index("s") * NC + lax.axis_index("c")
    base = wid * RPW
    idx16 = lax.iota(jnp.int32, 16)

    def start_fetch(tr, b):
        pltpu.async_copy(lutT.at[:, pl.ds(tr * 128, 128)], tbuf.at[b],
                         gsems[b])

    def wait_fetch(tr, b):
        pltpu.make_async_copy(lutT.at[:, pl.ds(tr * 128, 128)], tbuf.at[b],
                              gsems[b]).wait()

    def start_write(tr, b):
        pltpu.async_copy(obuf.at[b], lin.at[pl.ds(tr * 64, 64), :],
                         ssems[b])

    def wait_write(tr, b):
        pltpu.make_async_copy(obuf.at[b], lin.at[pl.ds(tr * 64, 64), :],
                              ssems[b]).wait()

    cvecs = [idx16 + (16 * m) for m in range(D_MODEL // LANES)]

    def transpose_scale(b):
        @pl.loop(0, 128, unroll=8)
        def _rows(r):
            rv = jnp.full((16,), r, jnp.int32)
            line = r // 2
            half = (r % 2) * 64
            vs = [plsc.load_gather(tbuf.at[b], [cvecs[m], rv])
                  for m in range(D_MODEL // LANES)]
            for m in range(D_MODEL // LANES):
                obuf[b, line, pl.ds(half + 16 * m, 16)] = vs[m] * SCALE

    for b in range(ABUF):
        start_fetch(base + b, b)

    for b in range(ABUF):
        tr = base + b
        wait_fetch(tr, b)
        transpose_scale(b)
        start_write(tr, b)
        start_fetch(tr + ABUF, b)

    @pl.loop(ABUF, RPW - ABUF, step=ABUF)
    def _main(g):
        for b in range(ABUF):
            tr = base + g + b
            wait_fetch(tr, b)
            wait_write(tr - ABUF, b)
            transpose_scale(b)
            start_write(tr, b)
            start_fetch(tr + ABUF, b)

    for b in range(ABUF):
        tr = base + RPW - ABUF + b
        wait_fetch(tr, b)
        wait_write(tr - ABUF, b)
        transpose_scale(b)
        start_write(tr, b)
    for b in range(ABUF):
        wait_write(base + RPW - ABUF + b, b)

    @pl.when(wid < EXTRA)
    def _extra():
        trx = NW * RPW + wid
        pltpu.sync_copy(lutT.at[:, pl.ds(trx * 128, 128)], tbuf.at[0])
        transpose_scale(0)
        pltpu.sync_copy(obuf.at[0], lin.at[pl.ds(trx * 64, 64), :])

    @pl.when(wid == EXTRA)
    def _tail():
        pltpu.sync_copy(tail, tailv)
        pltpu.sync_copy(tailv, lin.at[pl.ds(NR_FULL * 64, TAIL_ROWS // 2), :])


def _gather_body(x6, lin, q, xv, idx2, gbuf, qbuf, *sems):
    gsems = sems[:BBUF]
    ssems = sems[BBUF:]
    wid = lax.axis_index("s") * NC + lax.axis_index("c")
    idx16 = lax.iota(jnp.int32, 16)

    pltpu.sync_copy(x6.at[wid], xv)

    def prep_lines(j, b):
        for m in range(BATCH // LANES):
            iv = xv[j // 8, j % 8, pl.ds(16 * m, 16)]
            idx2[b, pl.ds(16 * m, 16)] = lax.shift_right_logical(iv, 1)

    def start_gather(j, b):
        prep_lines(j, b)
        pltpu.async_copy(lin.at[idx2.at[b]], gbuf.at[b], gsems[b])

    def wait_gather(j, b):
        pltpu.make_async_copy(lin.at[idx2.at[b]], gbuf.at[b],
                              gsems[b]).wait()

    def q_slice(j):
        return q.at[j, :, pl.ds(wid * 128, 128)]

    def start_scatter(j, b):
        pltpu.async_copy(qbuf.at[b], q_slice(j), ssems[b])

    def wait_scatter(j, b):
        pltpu.make_async_copy(qbuf.at[b], q_slice(j), ssems[b]).wait()

    def transpose_block(j, b):
        paroff = []
        for m in range(BATCH // LANES):
            iv = xv[j // 8, j % 8, pl.ds(16 * m, 16)]
            paroff.append(lax.shift_left(iv & 1, 6))

        svecs = [idx16 + (16 * m) for m in range(BATCH // LANES)]

        @pl.loop(0, D_MODEL, unroll=4)
        def _cols(d):
            dv = jnp.full((16,), d, jnp.int32)
            cols = [paroff[m] + dv for m in range(BATCH // LANES)]
            vs = [plsc.load_gather(gbuf.at[b], [svecs[m], cols[m]])
                  for m in range(BATCH // LANES)]
            for m in range(BATCH // LANES):
                qbuf[b, d, pl.ds(16 * m, 16)] = vs[m]

    for b in range(BBUF):
        start_gather(b, b)

    for b in range(BBUF):
        wait_gather(b, b)
        transpose_block(b, b)
        start_scatter(b, b)
        start_gather(b + BBUF, b)

    @pl.loop(BBUF, N_I1 - BBUF, step=BBUF)
    def _main(g):
        for b in range(BBUF):
            j = g + b
            wait_gather(j, b)
            wait_scatter(j - BBUF, b)
            transpose_block(j, b)
            start_scatter(j, b)
            start_gather(j + BBUF, b)

    for b in range(BBUF):
        j = N_I1 - BBUF + b
        wait_gather(j, b)
        wait_scatter(j - BBUF, b)
        transpose_block(j, b)
        start_scatter(j, b)
    for b in range(BBUF):
        wait_scatter(N_I1 - BBUF + b, b)


def kernel(x, lut):
    rows, cols = x.shape
    assert (rows, cols) == (4096, N_I1)
    assert lut.shape == (VOCAB, D_MODEL)

    mesh = plsc.VectorSubcoreMesh(
        core_axis_name="c", subcore_axis_name="s",
        num_cores=NC, num_subcores=NS)

    lutT = lut.T
    tail = (lut[NR_FULL * 128:] * SCALE).reshape(TAIL_ROWS // 2, 128)
    detile = pl.kernel(
        _detile_body,
        out_type=jax.ShapeDtypeStruct((VOCAB // 2, 128), jnp.float32),
        mesh=mesh,
        scratch_types=(
            [pltpu.VMEM((ABUF, 64, 128), jnp.float32),
             pltpu.VMEM((ABUF, 64, 128), jnp.float32),
             pltpu.VMEM((TAIL_ROWS // 2, 128), jnp.float32)]
            + [pltpu.SemaphoreType.DMA] * (2 * ABUF)
        ),
        compiler_params=pltpu.CompilerParams(
            use_tc_tiling_on_sc=True, needs_layout_passes=False),
    )
    lin = detile(lutT, tail)

    x6 = x.T.reshape(25, 8, 32, 128).transpose(2, 0, 1, 3)
    gather = pl.kernel(
        _gather_body,
        out_type=jax.ShapeDtypeStruct((N_I1, D_MODEL, 4096), jnp.float32),
        mesh=mesh,
        scratch_types=(
            [pltpu.VMEM((25, 8, 128), jnp.int32),
             pltpu.VMEM((BBUF, 128), jnp.int32),
             pltpu.VMEM((BBUF, BATCH, 128), jnp.float32),
             pltpu.VMEM((BBUF, D_MODEL, 128), jnp.float32)]
            + [pltpu.SemaphoreType.DMA] * (2 * BBUF)
        ),
        compiler_params=pltpu.CompilerParams(
            use_tc_tiling_on_sc=True, needs_layout_passes=False),
    )
    q = gather(x6, lin)

    return q.transpose(2, 0, 1)


# speedup vs baseline: 2.3483x; 1.8605x over previous
import functools
import math

import jax
import jax.numpy as jnp
from jax import lax
from jax.experimental import pallas as pl
from jax.experimental.pallas import tpu as pltpu
from jax.experimental.pallas import tpu_sc as plsc

D_MODEL = 64
SCALE = math.sqrt(D_MODEL)

NC = 2
NS = 16
NW = NC * NS
LANES = 16

BATCH = 128
NBUF = 4


def _scale_batch(src, dst, b):
    @functools.partial(plsc.parallel_loop, 0, BATCH, unroll=8)
    def _rows(r):
        for c in range(D_MODEL // LANES):
            sl = pl.ds(c * LANES, LANES)
            dst[b, r, sl] = src[b, r, sl] * SCALE


def _emb_body(nbatch, b_per_w, x_hbm, lut_hbm, out_hbm, idx_v, rows_g,
              rows_s, *sems):
    gsems = sems[:NBUF]
    ssems = sems[NBUF:]
    wid = lax.axis_index("s") * NC + lax.axis_index("c")
    base = wid * b_per_w

    pltpu.sync_copy(x_hbm.at[wid], idx_v)

    def start_gather(j, b):
        pltpu.async_copy(lut_hbm.at[idx_v.at[j]], rows_g.at[b], gsems[b])

    def wait_gather(j, b):
        pltpu.make_async_copy(
            lut_hbm.at[idx_v.at[j]], rows_g.at[b], gsems[b]).wait()

    def out_slice(j):
        return out_hbm.at[pl.ds(base + j * BATCH, BATCH)]

    def start_scatter(j, b):
        pltpu.async_copy(rows_s.at[b], out_slice(j), ssems[b])

    def wait_scatter(j, b):
        pltpu.make_async_copy(rows_s.at[b], out_slice(j), ssems[b]).wait()

    for b in range(NBUF):
        start_gather(b, b)

    for b in range(NBUF):
        wait_gather(b, b)
        _scale_batch(rows_g, rows_s, b)
        start_scatter(b, b)
        start_gather(b + NBUF, b)

    steady = nbatch - NBUF

    @pl.loop(NBUF, steady, step=NBUF)
    def _main(g):
        for b in range(NBUF):
            j = g + b
            wait_gather(j, b)
            wait_scatter(j - NBUF, b)
            _scale_batch(rows_g, rows_s, b)
            start_scatter(j, b)
            start_gather(j + NBUF, b)

    for b in range(NBUF):
        j = steady + b
        wait_gather(j, b)
        wait_scatter(j - NBUF, b)
        _scale_batch(rows_g, rows_s, b)
        start_scatter(j, b)
    for b in range(NBUF):
        wait_scatter(steady + b, b)


def kernel(x, lut):
    rows, cols = x.shape
    total = rows * cols
    assert total % (NW * BATCH) == 0
    b_per_w = total // NW
    nbatch = b_per_w // BATCH

    x_flat = x.reshape(NW, nbatch, BATCH)

    mesh = plsc.VectorSubcoreMesh(
        core_axis_name="c", subcore_axis_name="s",
        num_cores=NC, num_subcores=NS)

    run = pl.kernel(
        functools.partial(_emb_body, nbatch, b_per_w),
        out_type=jax.ShapeDtypeStruct((total, D_MODEL), jnp.float32),
        mesh=mesh,
        scratch_types=(
            [pltpu.VMEM((nbatch, BATCH), jnp.int32),
             pltpu.VMEM((NBUF, BATCH, D_MODEL), jnp.float32),
             pltpu.VMEM((NBUF, BATCH, D_MODEL), jnp.float32)]
            + [pltpu.SemaphoreType.DMA] * (2 * NBUF)
        ),
        compiler_params=pltpu.CompilerParams(use_tc_tiling_on_sc=False),
    )
    out = run(x_flat, lut)
    return out.reshape(rows, cols, D_MODEL)
